# Initial kernel scaffold; baseline (speedup 1.0000x reference)
import jax
import jax.numpy as jnp
from jax.experimental import pallas as pl


def kernel(x, word_embeddings_weight):
    raise NotImplementedError("write your pallas kernel here")


# speedup vs baseline: 1.9087x; 1.9087x over previous
import functools

import jax
import jax.numpy as jnp
from jax import lax
from jax.experimental import pallas as pl
from jax.experimental.pallas import tpu as pltpu
from jax.experimental.pallas import tpu_sc as plsc

VOCAB = 100000
HIDDEN = 128
BATCH = 1024
SEQ = 200
N_IDX = BATCH * SEQ
NW = 32
PER_W = N_IDX // NW
CHUNK = 128
N_CHUNK = PER_W // CHUNK


def _make_sc_gather():
    mesh = plsc.VectorSubcoreMesh(core_axis_name="c", subcore_axis_name="s")

    @functools.partial(
        pl.kernel,
        mesh=mesh,
        out_type=jax.ShapeDtypeStruct((N_IDX, HIDDEN), jnp.float32),
        scratch_types=[
            pltpu.VMEM((CHUNK,), jnp.int32),
            pltpu.VMEM((CHUNK, HIDDEN), jnp.float32),
            pltpu.SemaphoreType.DMA,
        ],
    )
    def gather_kernel(idx_hbm, table_hbm, out_hbm, idx_v, rows_v, sem):
        wid = lax.axis_index("s") * 2 + lax.axis_index("c")
        base = wid * PER_W

        def body(i, carry):
            off = base + i * CHUNK
            pltpu.sync_copy(idx_hbm.at[pl.ds(off, CHUNK)], idx_v)
            pltpu.async_copy(table_hbm.at[idx_v], rows_v, sem).wait()
            pltpu.sync_copy(rows_v, out_hbm.at[pl.ds(off, CHUNK)])
            return carry

        lax.fori_loop(0, N_CHUNK, body, 0)

    return gather_kernel


_sc_gather = _make_sc_gather()

_TR_BB = 8


def _transpose_body(g_ref, o_ref):
    o_ref[...] = jnp.transpose(g_ref[...], (0, 2, 1))


def _tc_transpose(g):
    return pl.pallas_call(
        _transpose_body,
        grid=(BATCH // _TR_BB,),
        in_specs=[pl.BlockSpec((_TR_BB, SEQ, HIDDEN), lambda i: (i, 0, 0))],
        out_specs=pl.BlockSpec((_TR_BB, HIDDEN, SEQ), lambda i: (i, 0, 0)),
        out_shape=jax.ShapeDtypeStruct((BATCH, HIDDEN, SEQ), jnp.float32),
    )(g)


def kernel(x, word_embeddings_weight):
    idx = x.reshape(N_IDX).astype(jnp.int32)
    gathered = _sc_gather(idx, word_embeddings_weight)
    return _tc_transpose(gathered.reshape(BATCH, SEQ, HIDDEN))


# speedup vs baseline: 8.0239x; 4.2039x over previous
import functools

import jax
import jax.numpy as jnp
from jax import lax
from jax.experimental import pallas as pl
from jax.experimental.pallas import tpu as pltpu
from jax.experimental.pallas import tpu_sc as plsc

VOCAB = 100000
HIDDEN = 128
BATCH = 1024
SEQ = 200
N_IDX = BATCH * SEQ
NW = 32
PER_W = N_IDX // NW
CHUNK = 80
N_CHUNK = PER_W // CHUNK
K = 5
N_GRP = N_CHUNK // K


def _make_sc_gather():
    mesh = plsc.VectorSubcoreMesh(core_axis_name="c", subcore_axis_name="s")

    @functools.partial(
        pl.kernel,
        mesh=mesh,
        out_type=jax.ShapeDtypeStruct((N_IDX, HIDDEN), jnp.float32),
        scratch_types=[
            pltpu.VMEM((N_CHUNK, CHUNK), jnp.int32),
            pltpu.VMEM((2 * K, CHUNK, HIDDEN), jnp.float32),
            pltpu.SemaphoreType.DMA,
            pltpu.SemaphoreType.DMA,
            pltpu.SemaphoreType.DMA,
            pltpu.SemaphoreType.DMA,
        ],
    )
    def gather_kernel(idx_hbm, table_hbm, out_hbm, idx_v, rows_v, sg0, sg1, so0, so1):
        sg = (sg0, sg1)
        so = (so0, so1)
        wid = lax.axis_index("s") * 2 + lax.axis_index("c")
        base = wid * PER_W
        pltpu.sync_copy(idx_hbm.at[wid], idx_v)

        def issue_gather(i, buf, p):
            pltpu.async_copy(table_hbm.at[idx_v.at[i]], rows_v.at[buf], sg[p])

        def wait_gather(i, buf, p):
            pltpu.make_async_copy(
                table_hbm.at[idx_v.at[i]], rows_v.at[buf], sg[p]
            ).wait()

        def issue_out(i, buf, p):
            pltpu.async_copy(
                rows_v.at[buf], out_hbm.at[pl.ds(base + i * CHUNK, CHUNK)], so[p]
            )

        def wait_out(i, buf, p):
            pltpu.make_async_copy(
                rows_v.at[buf], out_hbm.at[pl.ds(base + i * CHUNK, CHUNK)], so[p]
            ).wait()

        for b in range(K):
            issue_gather(b, b, 0)

        def body(j2, carry):
            for p in range(2):
                j = j2 * 2 + p
                for b in range(K):
                    i = j * K + b
                    wait_gather(i, p * K + b, p)
                    issue_out(i, p * K + b, p)
                @pl.when(j < N_GRP - 1)
                def _():
                    q = 1 - p
                    for b in range(K):
                        @pl.when(j > 0)
                        def _():
                            wait_out((j - 1) * K + b, q * K + b, q)
                        issue_gather((j + 1) * K + b, q * K + b, q)
            return carry

        lax.fori_loop(0, N_GRP // 2, body, 0)

        for p in range(2):
            j = N_GRP - 2 + p
            for b in range(K):
                wait_out(j * K + b, p * K + b, p)

    return gather_kernel


_sc_gather = _make_sc_gather()


def kernel(x, word_embeddings_weight):
    idx = x.reshape(NW, N_CHUNK, CHUNK).astype(jnp.int32)
    gathered = _sc_gather(idx, word_embeddings_weight)
    return jnp.transpose(gathered.reshape(BATCH, SEQ, HIDDEN), (0, 2, 1))
